# Initial kernel scaffold; baseline (speedup 1.0000x reference)
import jax
import jax.numpy as jnp
from jax.experimental import pallas as pl


def kernel(x, codebooks):
    raise NotImplementedError("write your pallas kernel here")


# speedup vs baseline: 4.0534x; 4.0534x over previous
import functools

import jax
import jax.numpy as jnp
from jax.experimental import pallas as pl
from jax.experimental.pallas import tpu as pltpu

_N_BOOKS = 16
_K = 256
_D = 32


def _vq_block_kernel(x_ref, cb_ref, x2_ref, c2_ref, codes_ref, recon_ref):
    x = x_ref[...]
    bt = x.shape[0]
    iota = jax.lax.broadcasted_iota(jnp.int32, (bt, _K), 1)
    code_cols = []
    recon_cols = []
    for n in range(_N_BOOKS):
        xn = x[:, n * _D:(n + 1) * _D]
        cn = cb_ref[n]
        cross = jax.lax.dot_general(
            xn, cn, (((1,), (1,)), ((), ())),
            preferred_element_type=jnp.float32)
        x2 = x2_ref[:, n:n + 1]
        c2_row = c2_ref[n:n + 1, :]
        score = jnp.sqrt(jnp.maximum((x2 + c2_row) - 2.0 * cross, 0.0))
        minval = jnp.min(score, axis=1, keepdims=True)
        idx = jnp.min(jnp.where(score == minval, iota, _K), axis=1,
                      keepdims=True)
        onehot = (iota == idx).astype(jnp.float32)
        rec = jax.lax.dot_general(
            onehot, cn, (((1,), (0,)), ((), ())),
            precision=jax.lax.Precision.HIGHEST,
            preferred_element_type=jnp.float32)
        code_cols.append(idx)
        recon_cols.append(rec)
    codes_ref[...] = jnp.concatenate(code_cols, axis=1)
    recon_ref[...] = jnp.concatenate(recon_cols, axis=1)


@functools.partial(jax.jit, static_argnames=("block_b",))
def _vq_tc(x, codebooks, block_b=1024):
    b, e = x.shape
    xr = x.reshape(b, _N_BOOKS, _D)
    x2 = jnp.sum(xr * xr, axis=-1)
    c2 = jnp.sum(codebooks * codebooks, axis=-1)
    grid = (b // block_b,)
    codes, recon = pl.pallas_call(
        _vq_block_kernel,
        grid=grid,
        in_specs=[
            pl.BlockSpec((block_b, e), lambda i: (i, 0)),
            pl.BlockSpec((_N_BOOKS, _K, _D), lambda i: (0, 0, 0)),
            pl.BlockSpec((block_b, _N_BOOKS), lambda i: (i, 0)),
            pl.BlockSpec((_N_BOOKS, _K), lambda i: (0, 0)),
        ],
        out_specs=[
            pl.BlockSpec((block_b, _N_BOOKS), lambda i: (i, 0)),
            pl.BlockSpec((block_b, e), lambda i: (i, 0)),
        ],
        out_shape=[
            jax.ShapeDtypeStruct((b, _N_BOOKS), jnp.int32),
            jax.ShapeDtypeStruct((b, e), jnp.float32),
        ],
    )(x, codebooks, x2, c2)
    return codes, recon


def kernel(x, codebooks):
    codes, recon = _vq_tc(x, codebooks)
    return codes.astype(jnp.uint8), recon


# speedup vs baseline: 12.2300x; 3.0173x over previous
import functools

import jax
import jax.numpy as jnp
from jax.experimental import pallas as pl
from jax.experimental.pallas import tpu as pltpu

_N_BOOKS = 16
_K = 256
_D = 32


def _vq_block_kernel(x_ref, cbm2_ref, cbhl_ref, c2_ref, codes_ref, recon_ref):
    x = x_ref[...]
    bt = x.shape[0]
    iota = jax.lax.broadcasted_iota(jnp.int32, (bt, _K), 1)
    code_cols = []
    recon_cols = []
    for n in range(_N_BOOKS):
        xn = x[:, n * _D:(n + 1) * _D]
        cross_m2 = jax.lax.dot_general(
            xn, cbm2_ref[n], (((1,), (1,)), ((), ())),
            preferred_element_type=jnp.float32)
        score = c2_ref[n:n + 1, :] + cross_m2
        minval = jnp.min(score, axis=1, keepdims=True)
        idx = jnp.min(jnp.where(score == minval, iota, _K), axis=1,
                      keepdims=True)
        onehot = (iota == idx).astype(jnp.float32)
        rec2 = jax.lax.dot_general(
            onehot, cbhl_ref[n], (((1,), (0,)), ((), ())),
            preferred_element_type=jnp.float32)
        code_cols.append(idx)
        recon_cols.append(rec2[:, :_D] + rec2[:, _D:])
    codes_ref[...] = jnp.concatenate(code_cols, axis=1)
    recon_ref[...] = jnp.concatenate(recon_cols, axis=1)


@functools.partial(jax.jit, static_argnames=("block_b",))
def _vq_tc(x, codebooks, block_b=1024):
    b, e = x.shape
    cbm2 = -2.0 * codebooks
    cb_hi = codebooks.astype(jnp.bfloat16).astype(jnp.float32)
    cbhl = jnp.concatenate([cb_hi, codebooks - cb_hi], axis=-1)
    c2 = jnp.sum(codebooks * codebooks, axis=-1)
    grid = (b // block_b,)
    codes, recon = pl.pallas_call(
        _vq_block_kernel,
        grid=grid,
        in_specs=[
            pl.BlockSpec((block_b, e), lambda i: (i, 0)),
            pl.BlockSpec((_N_BOOKS, _K, _D), lambda i: (0, 0, 0)),
            pl.BlockSpec((_N_BOOKS, _K, 2 * _D), lambda i: (0, 0, 0)),
            pl.BlockSpec((_N_BOOKS, _K), lambda i: (0, 0)),
        ],
        out_specs=[
            pl.BlockSpec((block_b, _N_BOOKS), lambda i: (i, 0)),
            pl.BlockSpec((block_b, e), lambda i: (i, 0)),
        ],
        out_shape=[
            jax.ShapeDtypeStruct((b, _N_BOOKS), jnp.int32),
            jax.ShapeDtypeStruct((b, e), jnp.float32),
        ],
    )(x, cbm2, cbhl, c2)
    return codes, recon


def kernel(x, codebooks):
    codes, recon = _vq_tc(x, codebooks)
    return codes.astype(jnp.uint8), recon
